# baseline (speedup 1.0000x reference)
import functools

import jax
import jax.numpy as jnp
from jax import lax
from jax.experimental import pallas as pl
from jax.experimental.pallas import tpu as pltpu
from jax.experimental.pallas import tpu_sc as plsc


def _sc_copy_1d(x):
    n = x.shape[0]
    info = plsc.get_sparse_core_info()
    nc, ns = info.num_cores, info.num_subcores
    nw = nc * ns
    per_w = n // nw
    nchunk = 2
    c = per_w // nchunk

    mesh = plsc.VectorSubcoreMesh(core_axis_name="c", subcore_axis_name="s")

    @functools.partial(
        pl.kernel,
        mesh=mesh,
        out_type=jax.ShapeDtypeStruct((n,), x.dtype),
        scratch_types=[
            pltpu.VMEM((nchunk, c), jnp.float32),
            pltpu.SemaphoreType.DMA((nchunk,)),
            pltpu.SemaphoreType.DMA((nchunk,)),
        ],
    )
    def sc_copy(x_hbm, o_hbm, buf, lsem, ssem):
        wid = lax.axis_index("s") * nc + lax.axis_index("c")
        base = wid * per_w

        def ld(j):
            return pltpu.make_async_copy(
                x_hbm.at[pl.ds(base + j * c, c)], buf.at[j], lsem.at[j])

        def st(j):
            return pltpu.make_async_copy(
                buf.at[j], o_hbm.at[pl.ds(base + j * c, c)], ssem.at[j])

        for j in range(nchunk):
            ld(j).start()
        for j in range(nchunk):
            ld(j).wait()
            st(j).start()
        for j in range(nchunk):
            st(j).wait()

    return sc_copy(x)


def kernel(x):
    squeeze_batch = False
    if x.ndim == 1:
        x = x[None, :]
        squeeze_batch = True
    b, t = x.shape
    out = _sc_copy_1d(x.reshape(-1)).reshape(b, t)
    if squeeze_batch:
        out = out[0]
    return out
